# baseline (speedup 1.0000x reference)
import functools
import math

import jax
import jax.numpy as jnp
from jax import lax
from jax.experimental import pallas as pl
from jax.experimental.pallas import tpu as pltpu
from jax.experimental.pallas import tpu_sc as plsc

N = 10000
D = 256
DH = 128
DQ = 64
S = 25
SP = 32
WQ = 80
E_REAL = 170000
E_PAD = 196608
CH = 1024
HC = 512
NCH = E_PAD // CH
NC = 2
NS = 16
K1_IT = NCH // (NC * NS)
K2_IT = NCH // NS
RCH = 16
NRCH = N // RCH
RIT = (NRCH + NS - 1) // NS
DK_INV = 1.0 / math.sqrt(S)

_mesh = plsc.VectorSubcoreMesh(
    core_axis_name="c", subcore_axis_name="s", num_cores=NC, num_subcores=NS)


BR = 1000


def _proj_body(x_ref, w_ref, b_ref, q_ref, k_ref,
               xa0_ref, xa1_ref, xa2_ref, xa3_ref):
    xv = x_ref[...]
    dn = (((1,), (0,)), ((), ()))
    qk = lax.dot_general(xv, w_ref[...], dn,
                         precision=lax.Precision.DEFAULT) + b_ref[...]
    q_ref[...] = jnp.tanh(qk[:, :SP]) * DK_INV
    k_ref[...] = qk[:, SP:]
    ones = jnp.ones((BR, 1), jnp.float32)
    zpad = jnp.zeros((BR, WQ - DQ - 1), jnp.float32)
    for t, ref in enumerate((xa0_ref, xa1_ref, xa2_ref, xa3_ref)):
        ref[...] = jnp.concatenate(
            [xv[:, t * DQ:(t + 1) * DQ], ones, zpad], axis=1)


_proj = pl.pallas_call(
    _proj_body,
    grid=(N // BR,),
    in_specs=[
        pl.BlockSpec((BR, D), lambda i: (i, 0)),
        pl.BlockSpec((D, 2 * SP), lambda i: (0, 0)),
        pl.BlockSpec((1, 2 * SP), lambda i: (0, 0)),
    ],
    out_specs=[
        pl.BlockSpec((BR, SP), lambda i: (i, 0)),
        pl.BlockSpec((BR, SP), lambda i: (i, 0)),
        pl.BlockSpec((BR, WQ), lambda i: (i, 0)),
        pl.BlockSpec((BR, WQ), lambda i: (i, 0)),
        pl.BlockSpec((BR, WQ), lambda i: (i, 0)),
        pl.BlockSpec((BR, WQ), lambda i: (i, 0)),
    ],
    out_shape=[
        jax.ShapeDtypeStruct((N, SP), jnp.float32),
        jax.ShapeDtypeStruct((N, SP), jnp.float32),
        jax.ShapeDtypeStruct((N, WQ), jnp.float32),
        jax.ShapeDtypeStruct((N, WQ), jnp.float32),
        jax.ShapeDtypeStruct((N, WQ), jnp.float32),
        jax.ShapeDtypeStruct((N, WQ), jnp.float32),
    ],
)


@functools.partial(
    pl.kernel,
    out_type=[
        jax.ShapeDtypeStruct((E_PAD, SP), jnp.float32),
        jax.ShapeDtypeStruct((E_PAD, SP), jnp.float32),
    ],
    mesh=_mesh,
    scratch_types=[
        pltpu.VMEM((CH,), jnp.int32),
        pltpu.VMEM((CH,), jnp.int32),
        pltpu.VMEM((CH, SP), jnp.float32),
        pltpu.VMEM((CH, SP), jnp.float32),
        pltpu.SemaphoreType.DMA,
    ],
    compiler_params=pltpu.CompilerParams(use_tc_tiling_on_sc=False),
)
def _edge_gather(q_hbm, k_hbm, src_hbm, dst_hbm, qg_hbm, kg_hbm,
                 sidx, didx, qg, kg, sem):
    c = lax.axis_index("c")
    s = lax.axis_index("s")
    wid = s * NC + c

    def chunk(i, _):
        ci = wid + (NC * NS) * i
        base = ci * CH

        @pl.when(base < E_REAL)
        def _():
            pltpu.sync_copy(src_hbm.at[pl.ds(base, CH)], sidx)
            pltpu.sync_copy(dst_hbm.at[pl.ds(base, CH)], didx)
            cps = []
            for j in range(CH // 128):
                cps.append(pltpu.async_copy(
                    q_hbm.at[sidx.at[pl.ds(j * 128, 128)]],
                    qg.at[pl.ds(j * 128, 128)], sem))
                cps.append(pltpu.async_copy(
                    k_hbm.at[didx.at[pl.ds(j * 128, 128)]],
                    kg.at[pl.ds(j * 128, 128)], sem))
            for cp in cps:
                cp.wait()
            pltpu.sync_copy(qg, qg_hbm.at[pl.ds(base, CH)])
            pltpu.sync_copy(kg, kg_hbm.at[pl.ds(base, CH)])
        return 0

    lax.fori_loop(0, K1_IT, chunk, 0)


PBLK = 4096


def _scores_body(qg_ref, kg_ref, p_ref):
    i = pl.program_id(0)
    w = jnp.sum(qg_ref[...] * kg_ref[...], axis=1)
    idx = i * PBLK + jax.lax.broadcasted_iota(jnp.int32, (PBLK,), 0)
    p_ref[...] = jnp.where(idx < E_REAL, jnp.exp(w), 0.0)


_scores = pl.pallas_call(
    _scores_body,
    grid=(E_PAD // PBLK,),
    in_specs=[
        pl.BlockSpec((PBLK, SP), lambda i: (i, 0)),
        pl.BlockSpec((PBLK, SP), lambda i: (i, 0)),
    ],
    out_specs=pl.BlockSpec((PBLK,), lambda i: (i,)),
    out_shape=jax.ShapeDtypeStruct((E_PAD,), jnp.float32),
)


@functools.partial(
    pl.kernel,
    out_type=jax.ShapeDtypeStruct((NC, N, DQ), jnp.float32),
    mesh=_mesh,
    scratch_types=[
        pltpu.VMEM((CH,), jnp.int32),
        pltpu.VMEM((CH,), jnp.int32),
        pltpu.VMEM((HC,), jnp.float32),
        pltpu.VMEM((HC, WQ), jnp.float32),
        pltpu.VMEM((RCH, WQ), jnp.float32),
        pltpu.VMEM((RCH, DQ), jnp.float32),
        pltpu.VMEM_SHARED((N, WQ), jnp.float32),
        pltpu.SemaphoreType.DMA,
    ],
    compiler_params=pltpu.CompilerParams(use_tc_tiling_on_sc=False),
)
def _accumulate(xa0_hbm, xa1_hbm, src_hbm, dst_hbm, p_hbm, out_hbm,
                sidx, didx, pb, xg, ob, oc, acc, sem):
    c = lax.axis_index("c")
    s = lax.axis_index("s")
    zeros16 = jnp.zeros((16,), jnp.float32)

    for r in range(RCH):
        for j in range(WQ // 16):
            ob[r, pl.ds(j * 16, 16)] = zeros16

    def zchunk(i, _):
        cid = s + NS * i

        @pl.when(cid < NRCH)
        def _():
            pltpu.sync_copy(ob, acc.at[pl.ds(cid * RCH, RCH)])
        return 0

    lax.fori_loop(0, RIT, zchunk, 0)
    plsc.subcore_barrier()

    def chunk(i, _):
        ci = s + NS * i
        base = ci * CH

        @pl.when(base < E_REAL)
        def _():
            pltpu.sync_copy(src_hbm.at[pl.ds(base, CH)], sidx)
            pltpu.sync_copy(dst_hbm.at[pl.ds(base, CH)], didx)
            for h in range(CH // HC):
                hbase = base + h * HC
                pltpu.sync_copy(p_hbm.at[pl.ds(hbase, HC)], pb)

                @pl.when(c == 0)
                def _():
                    cps = [pltpu.async_copy(
                        xa0_hbm.at[sidx.at[pl.ds(h * HC + j * 128, 128)]],
                        xg.at[pl.ds(j * 128, 128)], sem)
                        for j in range(HC // 128)]
                    for cp in cps:
                        cp.wait()

                @pl.when(c == 1)
                def _():
                    cps = [pltpu.async_copy(
                        xa1_hbm.at[sidx.at[pl.ds(h * HC + j * 128, 128)]],
                        xg.at[pl.ds(j * 128, 128)], sem)
                        for j in range(HC // 128)]
                    for cp in cps:
                        cp.wait()

                def scale(g, _):
                    pv = pb[pl.ds(g * 16, 16)]
                    for l in range(16):
                        e = g * 16 + l
                        pe = lax.broadcast(pv[l], (16,))
                        for j in range(WQ // 16):
                            xg[e, pl.ds(j * 16, 16)] = (
                                xg[e, pl.ds(j * 16, 16)] * pe)
                    return 0

                lax.fori_loop(0, HC // 16, scale, 0)
                for j in range(HC // 128):
                    pltpu.sync_copy(
                        xg.at[pl.ds(j * 128, 128)],
                        acc.at[didx.at[pl.ds(h * HC + j * 128, 128)]],
                        add=True)
        return 0

    lax.fori_loop(0, K2_IT, chunk, 0)
    plsc.subcore_barrier()

    def wchunk(i, _):
        cid = s + NS * i

        @pl.when(cid < NRCH)
        def _():
            rbase = cid * RCH
            pltpu.sync_copy(acc.at[pl.ds(rbase, RCH)], ob)

            def nrow(r, _):
                dv = ob[r, pl.ds(DQ, 16)]
                inv = 1.0 / lax.broadcast(dv[0], (16,))
                for j in range(DQ // 16):
                    oc[r, pl.ds(j * 16, 16)] = ob[r, pl.ds(j * 16, 16)] * inv
                return 0

            lax.fori_loop(0, RCH, nrow, 0)
            pltpu.sync_copy(oc, out_hbm.at[c, pl.ds(rbase, RCH)])
        return 0

    lax.fori_loop(0, RIT, wchunk, 0)


def kernel(x, edge_index, Wq, bq, Wk, bk):
    loop = jnp.arange(N, dtype=edge_index.dtype)
    zpad = jnp.zeros((E_PAD - E_REAL,), jnp.int32)
    src1 = jnp.concatenate(
        [edge_index[0].astype(jnp.int32), loop.astype(jnp.int32), zpad])
    dst1 = jnp.concatenate(
        [edge_index[1].astype(jnp.int32), loop.astype(jnp.int32), zpad])

    w = jnp.concatenate(
        [jnp.pad(Wq.T, ((0, 0), (0, SP - S))),
         jnp.pad(Wk.T, ((0, 0), (0, SP - S)))], axis=1)
    b = jnp.concatenate(
        [jnp.pad(bq, (0, SP - S)), jnp.pad(bk, (0, SP - S))]).reshape(1, 2 * SP)

    q32, k32, xa0, xa1, xa2, xa3 = _proj(x, w, b)
    qg, kg = _edge_gather(q32, k32, src1, dst1)
    p = _scores(qg, kg)
    oa = _accumulate(xa0, xa1, src1, dst1, p)
    ob_ = _accumulate(xa2, xa3, src1, dst1, p)
    return jnp.concatenate([oa[0], oa[1], ob_[0], ob_[1]], axis=1)


# speedup vs baseline: 1.1775x; 1.1775x over previous
import functools
import math

import jax
import jax.numpy as jnp
from jax import lax
from jax.experimental import pallas as pl
from jax.experimental.pallas import tpu as pltpu
from jax.experimental.pallas import tpu_sc as plsc

N = 10000
D = 256
DH = 128
DQ = 64
S = 25
SP = 32
WQ = 80
E_REAL = 170000
E_PAD = 196608
CH = 1024
HC = 512
NCH = E_PAD // CH
NC = 2
NS = 16
K1_IT = NCH // (NC * NS)
K2_IT = NCH // NS
RCH = 16
NRCH = N // RCH
RIT = (NRCH + NS - 1) // NS
DK_INV = 1.0 / math.sqrt(S)

_mesh = plsc.VectorSubcoreMesh(
    core_axis_name="c", subcore_axis_name="s", num_cores=NC, num_subcores=NS)


BR = 1000


def _proj_body(x_ref, w_ref, b_ref, q_ref, k_ref,
               xa0_ref, xa1_ref, xa2_ref, xa3_ref):
    xv = x_ref[...]
    dn = (((1,), (0,)), ((), ()))
    qk = lax.dot_general(xv, w_ref[...], dn,
                         precision=lax.Precision.DEFAULT) + b_ref[...]
    q_ref[...] = jnp.tanh(qk[:, :SP]) * DK_INV
    k_ref[...] = qk[:, SP:]
    ones = jnp.ones((BR, 1), jnp.float32)
    zpad = jnp.zeros((BR, WQ - DQ - 1), jnp.float32)
    for t, ref in enumerate((xa0_ref, xa1_ref, xa2_ref, xa3_ref)):
        ref[...] = jnp.concatenate(
            [xv[:, t * DQ:(t + 1) * DQ], ones, zpad], axis=1)


_proj = pl.pallas_call(
    _proj_body,
    grid=(N // BR,),
    in_specs=[
        pl.BlockSpec((BR, D), lambda i: (i, 0)),
        pl.BlockSpec((D, 2 * SP), lambda i: (0, 0)),
        pl.BlockSpec((1, 2 * SP), lambda i: (0, 0)),
    ],
    out_specs=[
        pl.BlockSpec((BR, SP), lambda i: (i, 0)),
        pl.BlockSpec((BR, SP), lambda i: (i, 0)),
        pl.BlockSpec((BR, WQ), lambda i: (i, 0)),
        pl.BlockSpec((BR, WQ), lambda i: (i, 0)),
        pl.BlockSpec((BR, WQ), lambda i: (i, 0)),
        pl.BlockSpec((BR, WQ), lambda i: (i, 0)),
    ],
    out_shape=[
        jax.ShapeDtypeStruct((N, SP), jnp.float32),
        jax.ShapeDtypeStruct((N, SP), jnp.float32),
        jax.ShapeDtypeStruct((N, WQ), jnp.float32),
        jax.ShapeDtypeStruct((N, WQ), jnp.float32),
        jax.ShapeDtypeStruct((N, WQ), jnp.float32),
        jax.ShapeDtypeStruct((N, WQ), jnp.float32),
    ],
)


@functools.partial(
    pl.kernel,
    out_type=[
        jax.ShapeDtypeStruct((E_PAD, SP), jnp.float32),
        jax.ShapeDtypeStruct((E_PAD, SP), jnp.float32),
    ],
    mesh=_mesh,
    scratch_types=[
        pltpu.VMEM((CH,), jnp.int32),
        pltpu.VMEM((CH,), jnp.int32),
        pltpu.VMEM((CH, SP), jnp.float32),
        pltpu.VMEM((CH, SP), jnp.float32),
        pltpu.SemaphoreType.DMA,
    ],
    compiler_params=pltpu.CompilerParams(use_tc_tiling_on_sc=False),
)
def _edge_gather(q_hbm, k_hbm, src_hbm, dst_hbm, qg_hbm, kg_hbm,
                 sidx, didx, qg, kg, sem):
    c = lax.axis_index("c")
    s = lax.axis_index("s")
    wid = s * NC + c

    def chunk(i, _):
        ci = wid + (NC * NS) * i
        base = ci * CH

        @pl.when(base < E_REAL)
        def _():
            pltpu.sync_copy(src_hbm.at[pl.ds(base, CH)], sidx)
            pltpu.sync_copy(dst_hbm.at[pl.ds(base, CH)], didx)
            cps = []
            for j in range(CH // 128):
                cps.append(pltpu.async_copy(
                    q_hbm.at[sidx.at[pl.ds(j * 128, 128)]],
                    qg.at[pl.ds(j * 128, 128)], sem))
                cps.append(pltpu.async_copy(
                    k_hbm.at[didx.at[pl.ds(j * 128, 128)]],
                    kg.at[pl.ds(j * 128, 128)], sem))
            for cp in cps:
                cp.wait()
            pltpu.sync_copy(qg, qg_hbm.at[pl.ds(base, CH)])
            pltpu.sync_copy(kg, kg_hbm.at[pl.ds(base, CH)])
        return 0

    lax.fori_loop(0, K1_IT, chunk, 0)


PBLK = 4096


EPB = PBLK // 128


def _scores_body(qg_ref, kg_ref, p_ref):
    i = pl.program_id(0)
    rows = PBLK * SP // 128
    qv = qg_ref[...].reshape(rows, 128)
    kv = kg_ref[...].reshape(rows, 128)
    prod = qv * kv
    lane = jax.lax.broadcasted_iota(jnp.int32, (128, 4), 0)
    col = jax.lax.broadcasted_iota(jnp.int32, (128, 4), 1)
    bd = jnp.where(lane // SP == col, 1.0, 0.0)
    w4 = lax.dot_general(prod, bd, (((1,), (0,)), ((), ())),
                         precision=lax.Precision.HIGHEST)
    idx = (i * PBLK
           + jax.lax.broadcasted_iota(jnp.int32, (rows, 4), 0) * 4
           + jax.lax.broadcasted_iota(jnp.int32, (rows, 4), 1))
    p_ref[...] = jnp.where(idx < E_REAL, jnp.exp(w4), 0.0)


_scores = pl.pallas_call(
    _scores_body,
    grid=(E_PAD // PBLK,),
    in_specs=[
        pl.BlockSpec((PBLK * SP,), lambda i: (i,)),
        pl.BlockSpec((PBLK * SP,), lambda i: (i,)),
    ],
    out_specs=pl.BlockSpec((PBLK * SP // 128, 4), lambda i: (i, 0)),
    out_shape=jax.ShapeDtypeStruct((E_PAD * SP // 128, 4), jnp.float32),
)


@functools.partial(
    pl.kernel,
    out_type=jax.ShapeDtypeStruct((NC, N, DQ), jnp.float32),
    mesh=_mesh,
    scratch_types=[
        pltpu.VMEM((CH,), jnp.int32),
        pltpu.VMEM((CH,), jnp.int32),
        pltpu.VMEM((HC,), jnp.float32),
        pltpu.VMEM((HC, WQ), jnp.float32),
        pltpu.VMEM((RCH, WQ), jnp.float32),
        pltpu.VMEM((RCH, DQ), jnp.float32),
        pltpu.VMEM_SHARED((N, WQ), jnp.float32),
        pltpu.SemaphoreType.DMA,
    ],
    compiler_params=pltpu.CompilerParams(use_tc_tiling_on_sc=False),
)
def _accumulate(xa0_hbm, xa1_hbm, src_hbm, dst_hbm, p_hbm, out_hbm,
                sidx, didx, pb, xg, ob, oc, acc, sem):
    c = lax.axis_index("c")
    s = lax.axis_index("s")
    zeros16 = jnp.zeros((16,), jnp.float32)

    for r in range(RCH):
        for j in range(WQ // 16):
            ob[r, pl.ds(j * 16, 16)] = zeros16

    def zchunk(i, _):
        cid = s + NS * i

        @pl.when(cid < NRCH)
        def _():
            pltpu.sync_copy(ob, acc.at[pl.ds(cid * RCH, RCH)])
        return 0

    lax.fori_loop(0, RIT, zchunk, 0)
    plsc.subcore_barrier()

    def chunk(i, _):
        ci = s + NS * i
        base = ci * CH

        @pl.when(base < E_REAL)
        def _():
            pltpu.sync_copy(src_hbm.at[pl.ds(base, CH)], sidx)
            pltpu.sync_copy(dst_hbm.at[pl.ds(base, CH)], didx)
            for h in range(CH // HC):
                hbase = base + h * HC
                pltpu.sync_copy(p_hbm.at[pl.ds(hbase, HC)], pb)

                @pl.when(c == 0)
                def _():
                    cps = [pltpu.async_copy(
                        xa0_hbm.at[sidx.at[pl.ds(h * HC + j * 128, 128)]],
                        xg.at[pl.ds(j * 128, 128)], sem)
                        for j in range(HC // 128)]
                    for cp in cps:
                        cp.wait()

                @pl.when(c == 1)
                def _():
                    cps = [pltpu.async_copy(
                        xa1_hbm.at[sidx.at[pl.ds(h * HC + j * 128, 128)]],
                        xg.at[pl.ds(j * 128, 128)], sem)
                        for j in range(HC // 128)]
                    for cp in cps:
                        cp.wait()

                def scale(g, _):
                    pv = pb[pl.ds(g * 16, 16)]
                    for l in range(16):
                        e = g * 16 + l
                        pe = lax.broadcast(pv[l], (16,))
                        for j in range(WQ // 16):
                            xg[e, pl.ds(j * 16, 16)] = (
                                xg[e, pl.ds(j * 16, 16)] * pe)
                    return 0

                lax.fori_loop(0, HC // 16, scale, 0)
                for j in range(HC // 128):
                    pltpu.sync_copy(
                        xg.at[pl.ds(j * 128, 128)],
                        acc.at[didx.at[pl.ds(h * HC + j * 128, 128)]],
                        add=True)
        return 0

    lax.fori_loop(0, K2_IT, chunk, 0)
    plsc.subcore_barrier()

    def wchunk(i, _):
        cid = s + NS * i

        @pl.when(cid < NRCH)
        def _():
            rbase = cid * RCH
            pltpu.sync_copy(acc.at[pl.ds(rbase, RCH)], ob)

            def nrow(r, _):
                dv = ob[r, pl.ds(DQ, 16)]
                inv = 1.0 / lax.broadcast(dv[0], (16,))
                for j in range(DQ // 16):
                    oc[r, pl.ds(j * 16, 16)] = ob[r, pl.ds(j * 16, 16)] * inv
                return 0

            lax.fori_loop(0, RCH, nrow, 0)
            pltpu.sync_copy(oc, out_hbm.at[c, pl.ds(rbase, RCH)])
        return 0

    lax.fori_loop(0, RIT, wchunk, 0)


def kernel(x, edge_index, Wq, bq, Wk, bk):
    loop = jnp.arange(N, dtype=edge_index.dtype)
    zpad = jnp.zeros((E_PAD - E_REAL,), jnp.int32)
    src1 = jnp.concatenate(
        [edge_index[0].astype(jnp.int32), loop.astype(jnp.int32), zpad])
    dst1 = jnp.concatenate(
        [edge_index[1].astype(jnp.int32), loop.astype(jnp.int32), zpad])

    w = jnp.concatenate(
        [jnp.pad(Wq.T, ((0, 0), (0, SP - S))),
         jnp.pad(Wk.T, ((0, 0), (0, SP - S)))], axis=1)
    b = jnp.concatenate(
        [jnp.pad(bq, (0, SP - S)), jnp.pad(bk, (0, SP - S))]).reshape(1, 2 * SP)

    q32, k32, xa0, xa1, xa2, xa3 = _proj(x, w, b)
    qg, kg = _edge_gather(q32, k32, src1, dst1)
    p = _scores(qg.reshape(E_PAD * SP), kg.reshape(E_PAD * SP)).reshape(E_PAD)
    oa = _accumulate(xa0, xa1, src1, dst1, p)
    ob_ = _accumulate(xa2, xa3, src1, dst1, p)
    return jnp.concatenate([oa[0], oa[1], ob_[0], ob_[1]], axis=1)


# speedup vs baseline: 1.3178x; 1.1192x over previous
import functools
import math

import jax
import jax.numpy as jnp
from jax import lax
from jax.experimental import pallas as pl
from jax.experimental.pallas import tpu as pltpu
from jax.experimental.pallas import tpu_sc as plsc

N = 10000
D = 256
DH = 128
DQ = 64
S = 25
SP = 32
WQ = 80
E_REAL = 170000
E_PAD = 196608
CH = 1024
HC = 256
NCH = E_PAD // CH
NC = 2
NS = 16
K1_IT = NCH // (NC * NS)
K2_IT = NCH // NS
RCH = 16
NRCH = N // RCH
RIT = (NRCH + NS - 1) // NS
DK_INV = 1.0 / math.sqrt(S)

_mesh = plsc.VectorSubcoreMesh(
    core_axis_name="c", subcore_axis_name="s", num_cores=NC, num_subcores=NS)


BR = 1000


def _proj_body(x_ref, w_ref, b_ref, q_ref, k_ref,
               xa0_ref, xa1_ref, xa2_ref, xa3_ref):
    xv = x_ref[...]
    dn = (((1,), (0,)), ((), ()))
    qk = lax.dot_general(xv, w_ref[...], dn,
                         precision=lax.Precision.DEFAULT) + b_ref[...]
    q_ref[...] = jnp.tanh(qk[:, :SP]) * DK_INV
    k_ref[...] = qk[:, SP:]
    ones = jnp.ones((BR, 1), jnp.float32)
    zpad = jnp.zeros((BR, WQ - DQ - 1), jnp.float32)
    for t, ref in enumerate((xa0_ref, xa1_ref, xa2_ref, xa3_ref)):
        ref[...] = jnp.concatenate(
            [xv[:, t * DQ:(t + 1) * DQ], ones, zpad], axis=1)


_proj = pl.pallas_call(
    _proj_body,
    grid=(N // BR,),
    in_specs=[
        pl.BlockSpec((BR, D), lambda i: (i, 0)),
        pl.BlockSpec((D, 2 * SP), lambda i: (0, 0)),
        pl.BlockSpec((1, 2 * SP), lambda i: (0, 0)),
    ],
    out_specs=[
        pl.BlockSpec((BR, SP), lambda i: (i, 0)),
        pl.BlockSpec((BR, SP), lambda i: (i, 0)),
        pl.BlockSpec((BR, WQ), lambda i: (i, 0)),
        pl.BlockSpec((BR, WQ), lambda i: (i, 0)),
        pl.BlockSpec((BR, WQ), lambda i: (i, 0)),
        pl.BlockSpec((BR, WQ), lambda i: (i, 0)),
    ],
    out_shape=[
        jax.ShapeDtypeStruct((N, SP), jnp.float32),
        jax.ShapeDtypeStruct((N, SP), jnp.float32),
        jax.ShapeDtypeStruct((N, WQ), jnp.float32),
        jax.ShapeDtypeStruct((N, WQ), jnp.float32),
        jax.ShapeDtypeStruct((N, WQ), jnp.float32),
        jax.ShapeDtypeStruct((N, WQ), jnp.float32),
    ],
)


@functools.partial(
    pl.kernel,
    out_type=[
        jax.ShapeDtypeStruct((E_PAD, SP), jnp.float32),
        jax.ShapeDtypeStruct((E_PAD, SP), jnp.float32),
    ],
    mesh=_mesh,
    scratch_types=[
        pltpu.VMEM((CH,), jnp.int32),
        pltpu.VMEM((CH,), jnp.int32),
        pltpu.VMEM((CH, SP), jnp.float32),
        pltpu.VMEM((CH, SP), jnp.float32),
        pltpu.SemaphoreType.DMA,
    ],
    compiler_params=pltpu.CompilerParams(use_tc_tiling_on_sc=False),
)
def _edge_gather(q_hbm, k_hbm, src_hbm, dst_hbm, qg_hbm, kg_hbm,
                 sidx, didx, qg, kg, sem):
    c = lax.axis_index("c")
    s = lax.axis_index("s")
    wid = s * NC + c

    def chunk(i, _):
        ci = wid + (NC * NS) * i
        base = ci * CH

        @pl.when(base < E_REAL)
        def _():
            pltpu.sync_copy(src_hbm.at[pl.ds(base, CH)], sidx)
            pltpu.sync_copy(dst_hbm.at[pl.ds(base, CH)], didx)
            cps = []
            for j in range(CH // 128):
                cps.append(pltpu.async_copy(
                    q_hbm.at[sidx.at[pl.ds(j * 128, 128)]],
                    qg.at[pl.ds(j * 128, 128)], sem))
                cps.append(pltpu.async_copy(
                    k_hbm.at[didx.at[pl.ds(j * 128, 128)]],
                    kg.at[pl.ds(j * 128, 128)], sem))
            for cp in cps:
                cp.wait()
            pltpu.sync_copy(qg, qg_hbm.at[pl.ds(base, CH)])
            pltpu.sync_copy(kg, kg_hbm.at[pl.ds(base, CH)])
        return 0

    lax.fori_loop(0, K1_IT, chunk, 0)


PBLK = 4096


EPB = PBLK // 128


def _scores_body(qg_ref, kg_ref, p_ref):
    i = pl.program_id(0)
    rows = PBLK * SP // 128
    qv = qg_ref[...].reshape(rows, 128)
    kv = kg_ref[...].reshape(rows, 128)
    prod = qv * kv
    lane = jax.lax.broadcasted_iota(jnp.int32, (128, 4), 0)
    col = jax.lax.broadcasted_iota(jnp.int32, (128, 4), 1)
    bd = jnp.where(lane // SP == col, 1.0, 0.0)
    w4 = lax.dot_general(prod, bd, (((1,), (0,)), ((), ())),
                         precision=lax.Precision.HIGHEST)
    idx = (i * PBLK
           + jax.lax.broadcasted_iota(jnp.int32, (rows, 4), 0) * 4
           + jax.lax.broadcasted_iota(jnp.int32, (rows, 4), 1))
    p_ref[...] = jnp.where(idx < E_REAL, jnp.exp(w4), 0.0)


_scores = pl.pallas_call(
    _scores_body,
    grid=(E_PAD // PBLK,),
    in_specs=[
        pl.BlockSpec((PBLK * SP,), lambda i: (i,)),
        pl.BlockSpec((PBLK * SP,), lambda i: (i,)),
    ],
    out_specs=pl.BlockSpec((PBLK * SP // 128, 4), lambda i: (i, 0)),
    out_shape=jax.ShapeDtypeStruct((E_PAD * SP // 128, 4), jnp.float32),
)


@functools.partial(
    pl.kernel,
    out_type=jax.ShapeDtypeStruct((NC, N, DQ), jnp.float32),
    mesh=_mesh,
    scratch_types=[
        pltpu.VMEM((CH,), jnp.int32),
        pltpu.VMEM((CH,), jnp.int32),
        pltpu.VMEM((CH,), jnp.float32),
        pltpu.VMEM((HC, WQ), jnp.float32),
        pltpu.VMEM((HC, WQ), jnp.float32),
        pltpu.VMEM((RCH, WQ), jnp.float32),
        pltpu.VMEM((RCH, DQ), jnp.float32),
        pltpu.VMEM_SHARED((N, WQ), jnp.float32),
        pltpu.SemaphoreType.DMA,
        pltpu.SemaphoreType.DMA,
    ],
    compiler_params=pltpu.CompilerParams(use_tc_tiling_on_sc=False),
)
def _accumulate(xa0_hbm, xa1_hbm, src_hbm, dst_hbm, p_hbm, out_hbm,
                sidx, didx, pb, xg0, xg1, ob, oc, acc, gsem0, gsem1):
    c = lax.axis_index("c")
    s = lax.axis_index("s")
    zeros16 = jnp.zeros((16,), jnp.float32)

    for r in range(RCH):
        for j in range(WQ // 16):
            ob[r, pl.ds(j * 16, 16)] = zeros16

    def zchunk(i, _):
        cid = s + NS * i

        @pl.when(cid < NRCH)
        def _():
            pltpu.sync_copy(ob, acc.at[pl.ds(cid * RCH, RCH)])
        return 0

    lax.fori_loop(0, RIT, zchunk, 0)
    plsc.subcore_barrier()

    def scale(buf, off):
        def grp(g, _):
            pv = pb[pl.ds(off + g * 16, 16)]
            for l in range(16):
                e = g * 16 + l
                pe = lax.broadcast(pv[l], (16,))
                for j in range(WQ // 16):
                    buf[e, pl.ds(j * 16, 16)] = buf[e, pl.ds(j * 16, 16)] * pe
            return 0
        lax.fori_loop(0, HC // 16, grp, 0)

    NH = CH // HC

    def issue(h):
        buf, sem_ = (xg0, gsem0) if h % 2 == 0 else (xg1, gsem1)

        @pl.when(c == 0)
        def _():
            for j in range(HC // 128):
                pltpu.async_copy(
                    xa0_hbm.at[sidx.at[pl.ds(h * HC + j * 128, 128)]],
                    buf.at[pl.ds(j * 128, 128)], sem_)

        @pl.when(c == 1)
        def _():
            for j in range(HC // 128):
                pltpu.async_copy(
                    xa1_hbm.at[sidx.at[pl.ds(h * HC + j * 128, 128)]],
                    buf.at[pl.ds(j * 128, 128)], sem_)

    def chunk(i, _):
        ci = s + NS * i
        base = ci * CH

        @pl.when(base < E_REAL)
        def _():
            pltpu.sync_copy(src_hbm.at[pl.ds(base, CH)], sidx)
            pltpu.sync_copy(dst_hbm.at[pl.ds(base, CH)], didx)
            pltpu.sync_copy(p_hbm.at[pl.ds(base, CH)], pb)
            issue(0)
            issue(1)
            for h in range(NH):
                buf, sem_ = (xg0, gsem0) if h % 2 == 0 else (xg1, gsem1)
                pltpu.make_async_copy(
                    xa0_hbm.at[pl.ds(0, HC)], buf, sem_).wait()
                scale(buf, h * HC)
                for j in range(HC // 128):
                    pltpu.sync_copy(
                        buf.at[pl.ds(j * 128, 128)],
                        acc.at[didx.at[pl.ds(h * HC + j * 128, 128)]],
                        add=True)
                if h + 2 < NH:
                    issue(h + 2)
        return 0

    lax.fori_loop(0, K2_IT, chunk, 0)
    plsc.subcore_barrier()

    def wchunk(i, _):
        cid = s + NS * i

        @pl.when(cid < NRCH)
        def _():
            rbase = cid * RCH
            pltpu.sync_copy(acc.at[pl.ds(rbase, RCH)], ob)

            def nrow(r, _):
                dv = ob[r, pl.ds(DQ, 16)]
                inv = 1.0 / lax.broadcast(dv[0], (16,))
                for j in range(DQ // 16):
                    oc[r, pl.ds(j * 16, 16)] = ob[r, pl.ds(j * 16, 16)] * inv
                return 0

            lax.fori_loop(0, RCH, nrow, 0)
            pltpu.sync_copy(oc, out_hbm.at[c, pl.ds(rbase, RCH)])
        return 0

    lax.fori_loop(0, RIT, wchunk, 0)


def kernel(x, edge_index, Wq, bq, Wk, bk):
    loop = jnp.arange(N, dtype=edge_index.dtype)
    zpad = jnp.zeros((E_PAD - E_REAL,), jnp.int32)
    src1 = jnp.concatenate(
        [edge_index[0].astype(jnp.int32), loop.astype(jnp.int32), zpad])
    dst1 = jnp.concatenate(
        [edge_index[1].astype(jnp.int32), loop.astype(jnp.int32), zpad])

    w = jnp.concatenate(
        [jnp.pad(Wq.T, ((0, 0), (0, SP - S))),
         jnp.pad(Wk.T, ((0, 0), (0, SP - S)))], axis=1)
    b = jnp.concatenate(
        [jnp.pad(bq, (0, SP - S)), jnp.pad(bk, (0, SP - S))]).reshape(1, 2 * SP)

    q32, k32, xa0, xa1, xa2, xa3 = _proj(x, w, b)
    qg, kg = _edge_gather(q32, k32, src1, dst1)
    p = _scores(qg.reshape(E_PAD * SP), kg.reshape(E_PAD * SP)).reshape(E_PAD)
    oa = _accumulate(xa0, xa1, src1, dst1, p)
    ob_ = _accumulate(xa2, xa3, src1, dst1, p)
    return jnp.concatenate([oa[0], oa[1], ob_[0], ob_[1]], axis=1)


# speedup vs baseline: 1.4777x; 1.1213x over previous
import functools
import math

import jax
import jax.numpy as jnp
from jax import lax
from jax.experimental import pallas as pl
from jax.experimental.pallas import tpu as pltpu
from jax.experimental.pallas import tpu_sc as plsc

N = 10000
D = 256
DH = 128
DQ = 64
S = 25
SP = 32
WQ = 80
E_REAL = 170000
E_PAD = 196608
CH = 1024
HC = 128
NCH = E_PAD // CH
NC = 2
NS = 16
K1_IT = NCH // (NC * NS)
K2_IT = NCH // NS
RCH = 16
NRCH = N // RCH
RIT = (NRCH + NS - 1) // NS
DK_INV = 1.0 / math.sqrt(S)

_mesh = plsc.VectorSubcoreMesh(
    core_axis_name="c", subcore_axis_name="s", num_cores=NC, num_subcores=NS)


BR = 1000


def _proj_body(x_ref, w_ref, b_ref, q_ref, k_ref,
               xa0_ref, xa1_ref, xa2_ref, xa3_ref):
    xv = x_ref[...]
    dn = (((1,), (0,)), ((), ()))
    qk = lax.dot_general(xv, w_ref[...], dn,
                         precision=lax.Precision.DEFAULT) + b_ref[...]
    q_ref[...] = jnp.tanh(qk[:, :SP]) * DK_INV
    k_ref[...] = qk[:, SP:]
    ones = jnp.ones((BR, 1), jnp.float32)
    zpad = jnp.zeros((BR, WQ - DQ - 1), jnp.float32)
    for t, ref in enumerate((xa0_ref, xa1_ref, xa2_ref, xa3_ref)):
        ref[...] = jnp.concatenate(
            [xv[:, t * DQ:(t + 1) * DQ], ones, zpad], axis=1)


_proj = pl.pallas_call(
    _proj_body,
    grid=(N // BR,),
    in_specs=[
        pl.BlockSpec((BR, D), lambda i: (i, 0)),
        pl.BlockSpec((D, 2 * SP), lambda i: (0, 0)),
        pl.BlockSpec((1, 2 * SP), lambda i: (0, 0)),
    ],
    out_specs=[
        pl.BlockSpec((BR, SP), lambda i: (i, 0)),
        pl.BlockSpec((BR, SP), lambda i: (i, 0)),
        pl.BlockSpec((BR, WQ), lambda i: (i, 0)),
        pl.BlockSpec((BR, WQ), lambda i: (i, 0)),
        pl.BlockSpec((BR, WQ), lambda i: (i, 0)),
        pl.BlockSpec((BR, WQ), lambda i: (i, 0)),
    ],
    out_shape=[
        jax.ShapeDtypeStruct((N, SP), jnp.float32),
        jax.ShapeDtypeStruct((N, SP), jnp.float32),
        jax.ShapeDtypeStruct((N, WQ), jnp.float32),
        jax.ShapeDtypeStruct((N, WQ), jnp.float32),
        jax.ShapeDtypeStruct((N, WQ), jnp.float32),
        jax.ShapeDtypeStruct((N, WQ), jnp.float32),
    ],
)


@functools.partial(
    pl.kernel,
    out_type=[
        jax.ShapeDtypeStruct((E_PAD, SP), jnp.float32),
        jax.ShapeDtypeStruct((E_PAD, SP), jnp.float32),
    ],
    mesh=_mesh,
    scratch_types=[
        pltpu.VMEM((CH,), jnp.int32),
        pltpu.VMEM((CH,), jnp.int32),
        pltpu.VMEM((CH, SP), jnp.float32),
        pltpu.VMEM((CH, SP), jnp.float32),
        pltpu.SemaphoreType.DMA,
    ],
    compiler_params=pltpu.CompilerParams(use_tc_tiling_on_sc=False),
)
def _edge_gather(q_hbm, k_hbm, src_hbm, dst_hbm, qg_hbm, kg_hbm,
                 sidx, didx, qg, kg, sem):
    c = lax.axis_index("c")
    s = lax.axis_index("s")
    wid = s * NC + c

    def chunk(i, _):
        ci = wid + (NC * NS) * i
        base = ci * CH

        @pl.when(base < E_REAL)
        def _():
            pltpu.sync_copy(src_hbm.at[pl.ds(base, CH)], sidx)
            pltpu.sync_copy(dst_hbm.at[pl.ds(base, CH)], didx)
            cps = []
            for j in range(CH // 128):
                cps.append(pltpu.async_copy(
                    q_hbm.at[sidx.at[pl.ds(j * 128, 128)]],
                    qg.at[pl.ds(j * 128, 128)], sem))
                cps.append(pltpu.async_copy(
                    k_hbm.at[didx.at[pl.ds(j * 128, 128)]],
                    kg.at[pl.ds(j * 128, 128)], sem))
            for cp in cps:
                cp.wait()
            pltpu.sync_copy(qg, qg_hbm.at[pl.ds(base, CH)])
            pltpu.sync_copy(kg, kg_hbm.at[pl.ds(base, CH)])
        return 0

    lax.fori_loop(0, K1_IT, chunk, 0)


PBLK = 4096


EPB = PBLK // 128


def _scores_body(qg_ref, kg_ref, p_ref):
    i = pl.program_id(0)
    rows = PBLK * SP // 128
    qv = qg_ref[...].reshape(rows, 128)
    kv = kg_ref[...].reshape(rows, 128)
    prod = qv * kv
    lane = jax.lax.broadcasted_iota(jnp.int32, (128, 4), 0)
    col = jax.lax.broadcasted_iota(jnp.int32, (128, 4), 1)
    bd = jnp.where(lane // SP == col, 1.0, 0.0)
    w4 = lax.dot_general(prod, bd, (((1,), (0,)), ((), ())),
                         precision=lax.Precision.HIGHEST)
    idx = (i * PBLK
           + jax.lax.broadcasted_iota(jnp.int32, (rows, 4), 0) * 4
           + jax.lax.broadcasted_iota(jnp.int32, (rows, 4), 1))
    p_ref[...] = jnp.where(idx < E_REAL, jnp.exp(w4), 0.0)


_scores = pl.pallas_call(
    _scores_body,
    grid=(E_PAD // PBLK,),
    in_specs=[
        pl.BlockSpec((PBLK * SP,), lambda i: (i,)),
        pl.BlockSpec((PBLK * SP,), lambda i: (i,)),
    ],
    out_specs=pl.BlockSpec((PBLK * SP // 128, 4), lambda i: (i, 0)),
    out_shape=jax.ShapeDtypeStruct((E_PAD * SP // 128, 4), jnp.float32),
)


@functools.partial(
    pl.kernel,
    out_type=jax.ShapeDtypeStruct((NC, 2, N, DQ), jnp.float32),
    mesh=_mesh,
    scratch_types=[
        pltpu.VMEM((CH,), jnp.int32),
        pltpu.VMEM((CH,), jnp.int32),
        pltpu.VMEM((CH,), jnp.float32),
        pltpu.VMEM((HC, WQ), jnp.float32),
        pltpu.VMEM((HC, WQ), jnp.float32),
        pltpu.VMEM((RCH, WQ), jnp.float32),
        pltpu.VMEM((RCH, DQ), jnp.float32),
        pltpu.VMEM_SHARED((N, WQ), jnp.float32),
        pltpu.VMEM_SHARED((N, WQ), jnp.float32),
        pltpu.SemaphoreType.DMA,
        pltpu.SemaphoreType.DMA,
    ],
    compiler_params=pltpu.CompilerParams(use_tc_tiling_on_sc=False),
)
def _accumulate(xa0_hbm, xa1_hbm, xa2_hbm, xa3_hbm, src_hbm, dst_hbm, p_hbm,
                out_hbm, sidx, didx, pb, xg0, xg1, ob, oc, accA, accB,
                gsem0, gsem1):
    c = lax.axis_index("c")
    s = lax.axis_index("s")
    zeros16 = jnp.zeros((16,), jnp.float32)

    for r in range(RCH):
        for j in range(WQ // 16):
            ob[r, pl.ds(j * 16, 16)] = zeros16

    def zchunk(i, _):
        cid = s + NS * i

        @pl.when(cid < NRCH)
        def _():
            pltpu.sync_copy(ob, accA.at[pl.ds(cid * RCH, RCH)])
            pltpu.sync_copy(ob, accB.at[pl.ds(cid * RCH, RCH)])
        return 0

    lax.fori_loop(0, RIT, zchunk, 0)
    plsc.subcore_barrier()

    def scale(buf, off):
        def grp(g, _):
            pv = pb[pl.ds(off + g * 16, 16)]
            for l in range(16):
                e = g * 16 + l
                pe = lax.broadcast(pv[l], (16,))
                for j in range(WQ // 16):
                    buf[e, pl.ds(j * 16, 16)] = buf[e, pl.ds(j * 16, 16)] * pe
            return 0
        lax.fori_loop(0, HC // 16, grp, 0)

    NU = 2 * (CH // HC)

    def issue(u):
        h, t = divmod(u, 2)
        buf, sem_ = (xg0, gsem0) if t == 0 else (xg1, gsem1)
        tabs0 = (xa0_hbm, xa2_hbm)
        tabs1 = (xa1_hbm, xa3_hbm)

        @pl.when(c == 0)
        def _():
            pltpu.async_copy(
                tabs0[t].at[sidx.at[pl.ds(h * HC, HC)]], buf, sem_)

        @pl.when(c == 1)
        def _():
            pltpu.async_copy(
                tabs1[t].at[sidx.at[pl.ds(h * HC, HC)]], buf, sem_)

    def chunk(i, _):
        ci = s + NS * i
        base = ci * CH

        @pl.when(base < E_REAL)
        def _():
            pltpu.sync_copy(src_hbm.at[pl.ds(base, CH)], sidx)
            pltpu.sync_copy(dst_hbm.at[pl.ds(base, CH)], didx)
            pltpu.sync_copy(p_hbm.at[pl.ds(base, CH)], pb)
            issue(0)
            issue(1)
            for u in range(NU):
                h, t = divmod(u, 2)
                buf, sem_ = (xg0, gsem0) if t == 0 else (xg1, gsem1)
                acc = accA if t == 0 else accB
                pltpu.make_async_copy(
                    xa0_hbm.at[pl.ds(0, HC)], buf, sem_).wait()
                scale(buf, h * HC)
                pltpu.sync_copy(
                    buf, acc.at[didx.at[pl.ds(h * HC, HC)]], add=True)
                if u + 2 < NU:
                    issue(u + 2)
        return 0

    lax.fori_loop(0, K2_IT, chunk, 0)
    plsc.subcore_barrier()

    def wchunk(i, _):
        cid = s + NS * i

        @pl.when(cid < NRCH)
        def _():
            rbase = cid * RCH
            for t in range(2):
                acc = accA if t == 0 else accB
                pltpu.sync_copy(acc.at[pl.ds(rbase, RCH)], ob)

                def nrow(r, _):
                    dv = ob[r, pl.ds(DQ, 16)]
                    inv = 1.0 / lax.broadcast(dv[0], (16,))
                    for j in range(DQ // 16):
                        oc[r, pl.ds(j * 16, 16)] = (
                            ob[r, pl.ds(j * 16, 16)] * inv)
                    return 0

                lax.fori_loop(0, RCH, nrow, 0)
                pltpu.sync_copy(oc, out_hbm.at[c, t, pl.ds(rbase, RCH)])
        return 0

    lax.fori_loop(0, RIT, wchunk, 0)


def kernel(x, edge_index, Wq, bq, Wk, bk):
    loop = jnp.arange(N, dtype=edge_index.dtype)
    zpad = jnp.zeros((E_PAD - E_REAL,), jnp.int32)
    src1 = jnp.concatenate(
        [edge_index[0].astype(jnp.int32), loop.astype(jnp.int32), zpad])
    dst1 = jnp.concatenate(
        [edge_index[1].astype(jnp.int32), loop.astype(jnp.int32), zpad])

    w = jnp.concatenate(
        [jnp.pad(Wq.T, ((0, 0), (0, SP - S))),
         jnp.pad(Wk.T, ((0, 0), (0, SP - S)))], axis=1)
    b = jnp.concatenate(
        [jnp.pad(bq, (0, SP - S)), jnp.pad(bk, (0, SP - S))]).reshape(1, 2 * SP)

    q32, k32, xa0, xa1, xa2, xa3 = _proj(x, w, b)
    qg, kg = _edge_gather(q32, k32, src1, dst1)
    p = _scores(qg.reshape(E_PAD * SP), kg.reshape(E_PAD * SP)).reshape(E_PAD)
    o = _accumulate(xa0, xa1, xa2, xa3, src1, dst1, p)
    return jnp.concatenate([o[0, 0], o[1, 0], o[0, 1], o[1, 1]], axis=1)
